# baseline (speedup 1.0000x reference)
import jax
import jax.numpy as jnp
from jax import lax
from jax.experimental import pallas as pl
from jax.experimental.pallas import tpu as pltpu

E = 8
K = 2
N = 2048
D = 768
F = 768
EP = 128
TN = 256
NTL = N // TN


def _moe_kernel(x32_ref, wg_ref, w1_ref, b1_ref, w2a_ref, b2p_ref,
                y_ref, loss_ref, h_ref, imp_ref, load_ref):
    t = pl.program_id(0)
    x32 = x32_ref[...]
    logits = jnp.dot(x32, wg_ref[...], preferred_element_type=jnp.float32)
    lane = lax.broadcasted_iota(jnp.int32, (TN, EP), 1)
    neg = jnp.float32(-jnp.inf)
    logits = jnp.where(lane < E, logits, neg)
    l1 = jnp.max(logits, axis=1, keepdims=True)
    a1 = jnp.min(jnp.where(logits == l1, lane, EP), axis=1, keepdims=True)
    m = jnp.where(lane == a1, neg, logits)
    l2 = jnp.max(m, axis=1, keepdims=True)
    a2 = jnp.min(jnp.where(m == l2, lane, EP), axis=1, keepdims=True)
    e2 = jnp.exp(l2 - l1)
    denom = 1.0 + e2
    g1 = 1.0 / denom
    g2 = e2 / denom
    gates = (jnp.where(lane == a1, g1, 0.0)
             + jnp.where(lane == a2, g2, 0.0))

    for e in range(E):
        h = jnp.dot(x32, w1_ref[e], preferred_element_type=jnp.float32)
        h = jnp.maximum(h + b1_ref[0, :, e * F:(e + 1) * F], 0.0)
        gcol = jnp.sum(jnp.where(lane == e, gates, 0.0), axis=1,
                       keepdims=True)
        h_ref[:, e * F:(e + 1) * F] = h * gcol

    o = jnp.dot(h_ref[...], w2a_ref[...], preferred_element_type=jnp.float32)
    o = o + jnp.dot(gates, b2p_ref[...], preferred_element_type=jnp.float32)
    y_ref[...] = o

    lane_m = (lane[0:1, :] < E).astype(jnp.float32)
    imp_t = jnp.sum(gates, axis=0, keepdims=True) * lane_m
    load_t = jnp.sum((gates > 0.0).astype(jnp.float32), axis=0,
                     keepdims=True) * lane_m

    @pl.when(t == 0)
    def _init():
        imp_ref[...] = imp_t
        load_ref[...] = load_t

    @pl.when(t > 0)
    def _acc():
        imp_ref[...] = imp_ref[...] + imp_t
        load_ref[...] = load_ref[...] + load_t

    @pl.when(t == NTL - 1)
    def _loss():
        def cv2(v):
            mean = jnp.sum(v) / E
            var = jnp.sum(jnp.where(lane_m > 0, (v - mean) ** 2,
                                    0.0)) / (E - 1)
            return var / (mean * mean + 1e-10)

        loss_ref[0, 0] = (cv2(imp_ref[...]) + cv2(load_ref[...])) * 0.01


@jax.jit
def _moe(data, w_gate_p, W1, b1a, W2a, b2p):
    y, loss = pl.pallas_call(
        _moe_kernel,
        grid=(NTL,),
        in_specs=[
            pl.BlockSpec((TN, D), lambda t: (t, 0)),
            pl.BlockSpec((D, EP), lambda t: (0, 0)),
            pl.BlockSpec((E, D, F), lambda t: (0, 0, 0)),
            pl.BlockSpec((1, 1, E * F), lambda t: (0, 0, 0)),
            pl.BlockSpec((E * F, D), lambda t: (0, 0)),
            pl.BlockSpec((EP, D), lambda t: (0, 0)),
        ],
        out_specs=[
            pl.BlockSpec((TN, D), lambda t: (t, 0)),
            pl.BlockSpec(memory_space=pltpu.SMEM),
        ],
        out_shape=[
            jax.ShapeDtypeStruct((N, D), jnp.float32),
            jax.ShapeDtypeStruct((1, 1), jnp.float32),
        ],
        scratch_shapes=[
            pltpu.VMEM((TN, E * F), jnp.float32),
            pltpu.VMEM((1, EP), jnp.float32),
            pltpu.VMEM((1, EP), jnp.float32),
        ],
        compiler_params=pltpu.CompilerParams(
            dimension_semantics=("arbitrary",),
        ),
    )(data, w_gate_p, W1, b1a, W2a, b2p)
    return y, loss[0, 0]


def kernel(data, w_gate, w_noise, W1, b1, W2, b2):
    del w_noise
    w_gate_p = jnp.pad(w_gate, ((0, 0), (0, EP - E)))
    b1a = b1.reshape(1, 1, E * F)
    W2a = W2.reshape(E * F, D)
    b2p = jnp.pad(b2, ((0, EP - E), (0, 0)))
    return _moe(data, w_gate_p, W1, b1a, W2a, b2p)


# speedup vs baseline: 1.3970x; 1.3970x over previous
import jax
import jax.numpy as jnp
from jax.experimental import pallas as pl
from jax.experimental.pallas import tpu as pltpu

E = 8
K = 2
N = 2048
D = 768
F = 768
EP = 128


def _moe_fused_kernel(data_ref, wg_ref, w1_ref, b1_ref, w2_ref, b2_ref,
                      y_ref, loss_ref, gates_ref):
    e = pl.program_id(0)

    @pl.when(e == 0)
    def _gating():
        x = data_ref[...]
        logits = jnp.dot(x, wg_ref[...], preferred_element_type=jnp.float32)
        lane = jax.lax.broadcasted_iota(jnp.int32, (N, EP), 1)
        neg = jnp.float32(-jnp.inf)
        logits = jnp.where(lane < E, logits, neg)
        l1 = jnp.max(logits, axis=1, keepdims=True)
        a1 = jnp.min(jnp.where(logits == l1, lane, EP), axis=1, keepdims=True)
        m = jnp.where(lane == a1, neg, logits)
        l2 = jnp.max(m, axis=1, keepdims=True)
        a2 = jnp.min(jnp.where(m == l2, lane, EP), axis=1, keepdims=True)
        e2 = jnp.exp(l2 - l1)
        denom = 1.0 + e2
        g1 = 1.0 / denom
        g2 = e2 / denom
        gates = (jnp.where(lane == a1, g1, 0.0)
                 + jnp.where(lane == a2, g2, 0.0))
        gates_ref[...] = gates
        lane_m = (lane[0:1, :] < E).astype(jnp.float32)
        importance = jnp.sum(gates, axis=0, keepdims=True) * lane_m
        load = jnp.sum((gates > 0.0).astype(jnp.float32), axis=0,
                       keepdims=True) * lane_m

        def cv2(v):
            mean = jnp.sum(v) / E
            var = jnp.sum(jnp.where(lane_m > 0, (v - mean) ** 2, 0.0)) / (E - 1)
            return var / (mean * mean + 1e-10)

        loss_ref[0, 0] = (cv2(importance) + cv2(load)) * 0.01

    x = data_ref[...]
    h = jnp.dot(x, w1_ref[0], preferred_element_type=jnp.float32)
    h = jnp.maximum(h + b1_ref[0], 0.0)
    o = jnp.dot(h, w2_ref[0], preferred_element_type=jnp.float32)
    o = o + b2_ref[0]
    lane = jax.lax.broadcasted_iota(jnp.int32, (N, EP), 1)
    gcol = jnp.sum(jnp.where(lane == e, gates_ref[...], 0.0), axis=1,
                   keepdims=True)
    contrib = o * gcol

    @pl.when(e == 0)
    def _init():
        y_ref[...] = contrib

    @pl.when(e > 0)
    def _acc():
        y_ref[...] = y_ref[...] + contrib


@jax.jit
def _moe_fused(data, w_gate_p, W1, b1, W2, b2):
    y, loss = pl.pallas_call(
        _moe_fused_kernel,
        grid=(E,),
        in_specs=[
            pl.BlockSpec((N, D), lambda e: (0, 0)),
            pl.BlockSpec((D, EP), lambda e: (0, 0)),
            pl.BlockSpec((1, D, F), lambda e: (e, 0, 0)),
            pl.BlockSpec((1, 1, F), lambda e: (e, 0, 0)),
            pl.BlockSpec((1, F, D), lambda e: (e, 0, 0)),
            pl.BlockSpec((1, 1, D), lambda e: (e, 0, 0)),
        ],
        out_specs=[
            pl.BlockSpec((N, D), lambda e: (0, 0)),
            pl.BlockSpec(memory_space=pltpu.SMEM),
        ],
        out_shape=[
            jax.ShapeDtypeStruct((N, D), jnp.float32),
            jax.ShapeDtypeStruct((1, 1), jnp.float32),
        ],
        scratch_shapes=[pltpu.VMEM((N, EP), jnp.float32)],
        compiler_params=pltpu.CompilerParams(
            dimension_semantics=("arbitrary",),
        ),
    )(data, w_gate_p, W1, b1, W2, b2)
    return y, loss[0, 0]


def kernel(data, w_gate, w_noise, W1, b1, W2, b2):
    del w_noise
    w_gate_p = jnp.pad(w_gate, ((0, 0), (0, EP - E)))
    return _moe_fused(data, w_gate_p, W1, b1[:, None, :], W2, b2[:, None, :])


# speedup vs baseline: 1.3999x; 1.0021x over previous
import jax
import jax.numpy as jnp
from jax.experimental import pallas as pl
from jax.experimental.pallas import tpu as pltpu

E = 8
K = 2
N = 2048
D = 768
F = 768
EP = 128
EPG = 2
NG = E // EPG


def _moe_fused_kernel(data_ref, wg_ref, w1_ref, b1_ref, w2_ref, b2_ref,
                      y_ref, loss_ref, gates_ref):
    e = pl.program_id(0)

    @pl.when(e == 0)
    def _gating():
        x = data_ref[...]
        logits = jnp.dot(x, wg_ref[...], preferred_element_type=jnp.float32)
        lane = jax.lax.broadcasted_iota(jnp.int32, (N, EP), 1)
        neg = jnp.float32(-jnp.inf)
        logits = jnp.where(lane < E, logits, neg)
        l1 = jnp.max(logits, axis=1, keepdims=True)
        a1 = jnp.min(jnp.where(logits == l1, lane, EP), axis=1, keepdims=True)
        m = jnp.where(lane == a1, neg, logits)
        l2 = jnp.max(m, axis=1, keepdims=True)
        a2 = jnp.min(jnp.where(m == l2, lane, EP), axis=1, keepdims=True)
        e2 = jnp.exp(l2 - l1)
        denom = 1.0 + e2
        g1 = 1.0 / denom
        g2 = e2 / denom
        gates = (jnp.where(lane == a1, g1, 0.0)
                 + jnp.where(lane == a2, g2, 0.0))
        gates_ref[...] = gates
        lane_m = (lane[0:1, :] < E).astype(jnp.float32)
        importance = jnp.sum(gates, axis=0, keepdims=True) * lane_m
        load = jnp.sum((gates > 0.0).astype(jnp.float32), axis=0,
                       keepdims=True) * lane_m

        def cv2(v):
            mean = jnp.sum(v) / E
            var = jnp.sum(jnp.where(lane_m > 0, (v - mean) ** 2, 0.0)) / (E - 1)
            return var / (mean * mean + 1e-10)

        loss_ref[0, 0] = (cv2(importance) + cv2(load)) * 0.01

    x = data_ref[...]
    lane = jax.lax.broadcasted_iota(jnp.int32, (N, EP), 1)
    contrib = None
    for k in range(EPG):
        h = jnp.dot(x, w1_ref[k], preferred_element_type=jnp.float32)
        h = jnp.maximum(h + b1_ref[k], 0.0)
        o = jnp.dot(h, w2_ref[k], preferred_element_type=jnp.float32)
        o = o + b2_ref[k]
        gcol = jnp.sum(jnp.where(lane == e * EPG + k, gates_ref[...], 0.0),
                       axis=1, keepdims=True)
        c = o * gcol
        contrib = c if contrib is None else contrib + c

    @pl.when(e == 0)
    def _init():
        y_ref[...] = contrib

    @pl.when(e > 0)
    def _acc():
        y_ref[...] = y_ref[...] + contrib


@jax.jit
def _moe_fused(data, w_gate_p, W1, b1, W2, b2):
    y, loss = pl.pallas_call(
        _moe_fused_kernel,
        grid=(NG,),
        in_specs=[
            pl.BlockSpec((N, D), lambda e: (0, 0)),
            pl.BlockSpec((D, EP), lambda e: (0, 0)),
            pl.BlockSpec((EPG, D, F), lambda e: (e, 0, 0)),
            pl.BlockSpec((EPG, 1, F), lambda e: (e, 0, 0)),
            pl.BlockSpec((EPG, F, D), lambda e: (e, 0, 0)),
            pl.BlockSpec((EPG, 1, D), lambda e: (e, 0, 0)),
        ],
        out_specs=[
            pl.BlockSpec((N, D), lambda e: (0, 0)),
            pl.BlockSpec(memory_space=pltpu.SMEM),
        ],
        out_shape=[
            jax.ShapeDtypeStruct((N, D), jnp.float32),
            jax.ShapeDtypeStruct((1, 1), jnp.float32),
        ],
        scratch_shapes=[pltpu.VMEM((N, EP), jnp.float32)],
        compiler_params=pltpu.CompilerParams(
            dimension_semantics=("arbitrary",),
        ),
    )(data, w_gate_p, W1, b1, W2, b2)
    return y, loss[0, 0]


def kernel(data, w_gate, w_noise, W1, b1, W2, b2):
    del w_noise
    w_gate_p = jnp.pad(w_gate, ((0, 0), (0, EP - E)))
    return _moe_fused(data, w_gate_p, W1, b1[:, None, :], W2, b2[:, None, :])
